# baseline (speedup 1.0000x reference)
import jax
import jax.numpy as jnp
from jax.experimental import pallas as pl
from jax.experimental.pallas import tpu as pltpu


def _xattn_kernel(in_ref, a_ref, cv_ref, ce_ref, out_ref):
    x = in_ref[0, 0]
    e = in_ref[0, 2]

    out_ws = jnp.dot(x, a_ref[...], preferred_element_type=jnp.float32)
    vexp = (jnp.dot(x, cv_ref[...], preferred_element_type=jnp.float32)
            + jnp.dot(e, ce_ref[...], preferred_element_type=jnp.float32))
    rl = pltpu.roll(vexp, 3, 1)
    rs = pltpu.roll(rl, 1, 0)
    lane = jax.lax.broadcasted_iota(jnp.int32, vexp.shape, 1)
    w = jnp.where(lane < 3, rs, rl)

    out_ref[0, 0] = out_ws + w
    out_ref[0, 1] = in_ref[0, 1]
    out_ref[0, 2] = in_ref[0, 2]


def kernel(input_data, Wq, Wk, Wv, Wo, Wself):
    B, C, N, D = input_data.shape
    width = 750
    rows = (N * D) // width
    nodes_per_row = width // D

    eye = jnp.eye(nodes_per_row, dtype=jnp.float32)
    a_mat = jnp.kron(eye, Wself)
    cv_mat = jnp.kron(eye, Wv[:D] @ Wo)
    ce_mat = jnp.kron(eye, Wv[D:] @ Wo)

    xr = input_data.reshape(B, C, rows, width)
    out = pl.pallas_call(
        _xattn_kernel,
        grid=(B,),
        in_specs=[
            pl.BlockSpec((1, C, rows, width), lambda b: (b, 0, 0, 0)),
            pl.BlockSpec((width, width), lambda b: (0, 0)),
            pl.BlockSpec((width, width), lambda b: (0, 0)),
            pl.BlockSpec((width, width), lambda b: (0, 0)),
        ],
        out_specs=pl.BlockSpec((1, C, rows, width), lambda b: (b, 0, 0, 0)),
        out_shape=jax.ShapeDtypeStruct((B, C, rows, width), jnp.float32),
    )(xr, a_mat, cv_mat, ce_mat)
    return out.reshape(B, C, N, D)


# speedup vs baseline: 4.2870x; 4.2870x over previous
import jax
import jax.numpy as jnp
from jax.experimental import pallas as pl
from jax.experimental.pallas import tpu as pltpu


def _xattn_kernel(in_ref, xt_ref, et_ref, wv_ref, wo_ref, ws_ref, out_ref):
    nb = pl.num_programs(1)
    j = pl.program_id(1)
    x = in_ref[0, 0]
    e = in_ref[0, 2]
    wv = wv_ref[...]

    v = (jnp.dot(x, wv[:3], preferred_element_type=jnp.float32)
         + jnp.dot(e, wv[3:], preferred_element_type=jnp.float32))

    jp = jnp.where(j == 0, nb - 1, j - 1)
    xt = xt_ref[0, pl.ds(jp, 1), :]
    et = et_ref[0, pl.ds(jp, 1), :]
    vprev = (jnp.dot(xt, wv[:3], preferred_element_type=jnp.float32)
             + jnp.dot(et, wv[3:], preferred_element_type=jnp.float32))

    rolled = pltpu.roll(v, 1, 0)
    row = jax.lax.broadcasted_iota(jnp.int32, rolled.shape, 0)
    vb = jnp.where(row == 0, jnp.broadcast_to(vprev, rolled.shape), rolled)

    out0 = (jnp.dot(x, ws_ref[...], preferred_element_type=jnp.float32)
            + jnp.dot(vb, wo_ref[...], preferred_element_type=jnp.float32))
    out_ref[0, 0] = out0
    out_ref[0, 1] = in_ref[0, 1]
    out_ref[0, 2] = in_ref[0, 2]


def kernel(input_data, Wq, Wk, Wv, Wo, Wself):
    B, C, N, D = input_data.shape
    R = 5000
    nb = N // R

    xt = input_data[:, 0, R - 1::R, :]
    et = input_data[:, 2, R - 1::R, :]

    return pl.pallas_call(
        _xattn_kernel,
        grid=(B, nb),
        in_specs=[
            pl.BlockSpec((1, C, R, D), lambda b, j: (b, 0, j, 0)),
            pl.BlockSpec((1, nb, D), lambda b, j: (b, 0, 0)),
            pl.BlockSpec((1, nb, D), lambda b, j: (b, 0, 0)),
            pl.BlockSpec((2 * D, 1), lambda b, j: (0, 0)),
            pl.BlockSpec((1, D), lambda b, j: (0, 0)),
            pl.BlockSpec((D, D), lambda b, j: (0, 0)),
        ],
        out_specs=pl.BlockSpec((1, C, R, D), lambda b, j: (b, 0, j, 0)),
        out_shape=jax.ShapeDtypeStruct((B, C, N, D), jnp.float32),
    )(input_data, xt, et, Wv, Wo, Wself)
